# baseline (speedup 1.0000x reference)
import jax
import jax.numpy as jnp
from jax.experimental import pallas as pl


def kernel(x, edge_index, edge_weight, params):
    h = x @ params['W_in'] + params['b_in']
    row = edge_index[0]
    col = edge_index[1]
    for i in range(3):
        h_self = h @ params[f'W_self_{i}'] + params[f'b_self_{i}']
        h_neigh = h @ params[f'W_neigh_{i}'] + params[f'b_neigh_{i}']
        edge_feat = edge_weight[:, None] * params[f'w_edge_{i}'][None, :]
        messages = h_neigh[col] * jax.nn.sigmoid(edge_feat)
        agg = jnp.zeros_like(h_self).at[row].add(messages)
        out = h_self + agg
        mu = jnp.mean(out, axis=-1, keepdims=True)
        var = jnp.var(out, axis=-1, keepdims=True)
        out = (out - mu) / jnp.sqrt(var + 1e-5) * params[f'ln_g_{i}'] + params[f'ln_b_{i}']
        out = jnp.where(out > 0, out, 0.2 * out)
        h = h + out
    return h


# speedup vs baseline: 2.2484x; 2.2484x over previous
import functools

import jax
import jax.numpy as jnp
from jax import lax
from jax.experimental import pallas as pl
from jax.experimental.pallas import tpu as pltpu
from jax.experimental.pallas import tpu_sc as plsc

H = 64
HH = 32
HP = 128
SUBS = 16
SUB = 128
GPC = 8
CHUNK = SUB * GPC
HSUB = 4
HALF = SUB * HSUB


def _mm_body(x_ref, w_ref, b_ref, o_ref):
    o_ref[...] = jnp.dot(x_ref[...], w_ref[...],
                         preferred_element_type=jnp.float32) + b_ref[...]


def _tc_matmul(h, w, b, rb):
    n, kdim = h.shape
    m = w.shape[1]
    nb = n // rb
    return pl.pallas_call(
        _mm_body,
        grid=(nb,),
        in_specs=[pl.BlockSpec((rb, kdim), lambda i: (i, 0)),
                  pl.BlockSpec((kdim, m), lambda i: (0, 0)),
                  pl.BlockSpec((1, m), lambda i: (0, 0))],
        out_specs=pl.BlockSpec((rb, m), lambda i: (i, 0)),
        out_shape=jax.ShapeDtypeStruct((n, m), jnp.float32),
    )(h, w, b.reshape(1, m))


def _mm3_body(x_ref, w_ref, b_ref, o_ref):
    o_ref[...] = jnp.dot(x_ref[...], w_ref[0],
                         preferred_element_type=jnp.float32) + b_ref[0]


def _tc_neigh(h, w, b, rb):
    n = h.shape[0]
    nb = n // rb
    w2 = w.reshape(H, 2, HH).transpose(1, 0, 2)
    b2 = b.reshape(2, 1, HH)
    return pl.pallas_call(
        _mm3_body,
        grid=(2, nb),
        in_specs=[pl.BlockSpec((rb, H), lambda c, i: (i, 0)),
                  pl.BlockSpec((1, H, HH), lambda c, i: (c, 0, 0)),
                  pl.BlockSpec((1, 1, HH), lambda c, i: (c, 0, 0))],
        out_specs=pl.BlockSpec((rb, HH), lambda c, i: (c * nb + i, 0)),
        out_shape=jax.ShapeDtypeStruct((2 * n, HH), jnp.float32),
    )(h, w2, b2)


def _post_body(h_ref, hs_ref, alo_ref, ahi_ref, g_ref, b_ref, o_ref):
    v = hs_ref[...] + jnp.concatenate([alo_ref[...], ahi_ref[...]], axis=1)
    mu = jnp.mean(v, axis=-1, keepdims=True)
    var = jnp.mean((v - mu) ** 2, axis=-1, keepdims=True)
    y = (v - mu) * lax.rsqrt(var + 1e-5) * g_ref[...] + b_ref[...]
    y = jnp.where(y > 0, y, 0.2 * y)
    o_ref[...] = h_ref[...] + y


def _tc_post(h, hs, alo, ahi, g, b, rb):
    n = h.shape[0]
    nb = n // rb
    return pl.pallas_call(
        _post_body,
        grid=(nb,),
        in_specs=[pl.BlockSpec((rb, H), lambda i: (i, 0)),
                  pl.BlockSpec((rb, H), lambda i: (i, 0)),
                  pl.BlockSpec((rb, HH), lambda i: (i, 0)),
                  pl.BlockSpec((rb, HH), lambda i: (i, 0)),
                  pl.BlockSpec((1, H), lambda i: (0, 0)),
                  pl.BlockSpec((1, H), lambda i: (0, 0))],
        out_specs=pl.BlockSpec((rb, H), lambda i: (i, 0)),
        out_shape=jax.ShapeDtypeStruct((n, H), jnp.float32),
    )(h, hs, alo, ahi, g.reshape(1, H), b.reshape(1, H))


def _sc_agg(hn2, colc, rowc, ewc, negw2, n_nodes, nch):
    r0s = -(-(n_nodes + 1) // SUBS)
    r0s = -(-r0s // 8) * 8
    agg_rows = SUBS * r0s
    mesh = plsc.VectorSubcoreMesh(core_axis_name="c", subcore_axis_name="s")

    @functools.partial(
        pl.kernel, mesh=mesh,
        out_type=(jax.ShapeDtypeStruct((agg_rows, HH), jnp.float32),
                  jax.ShapeDtypeStruct((agg_rows, HH), jnp.float32)),
        compiler_params=pltpu.CompilerParams(use_tc_tiling_on_sc=False,
                                             needs_layout_passes=False),
        scratch_types=[
            pltpu.VMEM((HALF, HH), jnp.float32),
            pltpu.VMEM((CHUNK,), jnp.int32),
            pltpu.VMEM((CHUNK,), jnp.int32),
            pltpu.VMEM((CHUNK // 16, 16), jnp.float32),
            pltpu.VMEM((HP,), jnp.float32),
            pltpu.VMEM((16,), jnp.int32),
            pltpu.VMEM_SHARED((agg_rows, HH), jnp.float32),
            pltpu.SemaphoreType.DMA,
        ],
    )
    def k(hn_hbm, col_hbm, row_hbm, ew_hbm, nw_hbm, out0_hbm, out1_hbm,
          rows_v, col_v, row_v, ew_v, nw_v, idx_v, agg_sh, sem):
        c = lax.axis_index("c")
        s = lax.axis_index("s")
        zero16 = jnp.zeros((16,), jnp.float32)

        @pl.loop(0, HALF)
        def _(r):
            rows_v[r, pl.ds(0, 16)] = zero16
            rows_v[r, pl.ds(16, 16)] = zero16

        zb = pl.multiple_of(s * r0s, 8)
        nfull = r0s // HALF
        rem = r0s - nfull * HALF
        for kk in range(nfull):
            pltpu.sync_copy(rows_v, agg_sh.at[pl.ds(zb + kk * HALF, HALF), :])
        if rem:
            pltpu.sync_copy(rows_v.at[pl.ds(0, rem), :],
                            agg_sh.at[pl.ds(zb + nfull * HALF, rem), :])
        pltpu.sync_copy(nw_hbm.at[c], nw_v)
        plsc.subcore_barrier()

        @pl.loop(0, nch)
        def _(ch):
            sci = s * nch + ch
            pltpu.sync_copy(col_hbm.at[c, sci], col_v)
            pltpu.sync_copy(row_hbm.at[sci], row_v)
            pltpu.sync_copy(ew_hbm.at[sci], ew_v)
            for half in range(2):
                pltpu.async_copy(
                    hn_hbm.at[col_v.at[pl.ds(half * HALF, HALF)]],
                    rows_v, sem).wait()

                idx_v[...] = jnp.full((16,), half * HALF, jnp.int32)

                @pl.loop(0, HALF)
                def _(r):
                    idx16 = idx_v[...]
                    w16 = plsc.load_gather(
                        ew_v, [lax.shift_right_logical(idx16, 4),
                               lax.bitwise_and(idx16, 15)])
                    d0 = jnp.exp(nw_v[pl.ds(0, 16)] * w16) + 1.0
                    d1 = jnp.exp(nw_v[pl.ds(16, 16)] * w16) + 1.0
                    rows_v[r, pl.ds(0, 16)] = rows_v[r, pl.ds(0, 16)] / d0
                    rows_v[r, pl.ds(16, 16)] = rows_v[r, pl.ds(16, 16)] / d1
                    idx_v[...] = idx16 + 1

                pltpu.sync_copy(rows_v,
                                agg_sh.at[row_v.at[pl.ds(half * HALF, HALF)]],
                                add=True)

        plsc.subcore_barrier()

        @pl.when(c == 0)
        def _():
            pltpu.sync_copy(agg_sh.at[pl.ds(zb, r0s), :],
                            out0_hbm.at[pl.ds(zb, r0s), :])

        @pl.when(c == 1)
        def _():
            pltpu.sync_copy(agg_sh.at[pl.ds(zb, r0s), :],
                            out1_hbm.at[pl.ds(zb, r0s), :])

    return k(hn2, colc, rowc, ewc, negw2)


def kernel(x, edge_index, edge_weight, params):
    n = x.shape[0]
    e = edge_weight.shape[0]
    nlayers = sum(1 for kk in params if kk.startswith('W_self_'))

    e_pad = SUBS * CHUNK * (-(-e // (SUBS * CHUNK)))
    per_sub = e_pad // SUBS
    nch = per_sub // CHUNK
    pad = e_pad - e

    row = edge_index[0]
    col = edge_index[1]
    ns = e_pad // CHUNK
    rowc = jnp.pad(row, (0, pad), constant_values=n).reshape(ns, CHUNK)
    colf = jnp.pad(col, (0, pad))
    colc = jnp.stack([colf, colf + n]).reshape(2, ns, CHUNK)
    ewc = jnp.pad(edge_weight, (0, pad)).reshape(ns, CHUNK // 16, 16)

    rb = 2000
    h = _tc_matmul(x, params['W_in'], params['b_in'], rb)
    for i in range(nlayers):
        hn2 = _tc_neigh(h, params[f'W_neigh_{i}'], params[f'b_neigh_{i}'], rb)
        negw2 = jnp.pad((-params[f'w_edge_{i}']).reshape(2, HH),
                        ((0, 0), (0, HP - HH)))
        alo, ahi = _sc_agg(hn2, colc, rowc, ewc, negw2, n, nch)
        hs = _tc_matmul(h, params[f'W_self_{i}'], params[f'b_self_{i}'], rb)
        h = _tc_post(h, hs, alo, ahi, params[f'ln_g_{i}'], params[f'ln_b_{i}'], rb)
    return h
